# baseline (speedup 1.0000x reference)
import jax
import jax.numpy as jnp
from jax.experimental import pallas as pl
from jax.experimental.pallas import tpu as pltpu

_B, _C, _H, _W = 64, 3, 224, 224
_BK = 16


def _scale_body(p_ref, m_ref, x_ref, o_ref):
    i = pl.program_id(0)
    for j in range(_BK):
        b = i * _BK + j
        scale = jnp.where(p_ref[b] != 0, m_ref[b], jnp.float32(1.0))
        o_ref[j] = x_ref[j] * scale


def kernel(input, probs, magnitudes):
    p = probs.astype(jnp.int32)
    return pl.pallas_call(
        _scale_body,
        grid=(_B // _BK,),
        in_specs=[
            pl.BlockSpec(memory_space=pltpu.SMEM),
            pl.BlockSpec(memory_space=pltpu.SMEM),
            pl.BlockSpec((_BK, _C, _H, _W), lambda i: (i, 0, 0, 0)),
        ],
        out_specs=pl.BlockSpec((_BK, _C, _H, _W), lambda i: (i, 0, 0, 0)),
        out_shape=jax.ShapeDtypeStruct((_B, _C, _H, _W), jnp.float32),
    )(p, magnitudes, input)
